# baseline (speedup 1.0000x reference)
import functools

import jax
import jax.numpy as jnp
from jax.experimental import pallas as pl
from jax.experimental.pallas import tpu as pltpu

_EPS = 1e-12


def _prep_body(g1c_ref, b1_ref, wd_ref, bd_ref, mem_ref, wk_ref, bk_ref,
               wv_ref, bv_ref, g3c_ref, b3_ref, wu_ref, bu_ref,
               wd_aug_ref, wu_aug_ref, key_ref, val_ref, const_ref):
    H, D = wd_ref.shape

    wdg = wd_ref[...] * g1c_ref[...]
    wd_aug_ref[...] = jnp.concatenate(
        [wdg, jnp.ones((H, 1), jnp.float32)], axis=1)

    wug = wu_ref[...] * g3c_ref[...]
    bias_row = jnp.dot(b3_ref[...], wu_ref[...],
                       preferred_element_type=jnp.float32) + bu_ref[...]
    wu_aug_ref[...] = jnp.concatenate([wug, bias_row], axis=0)

    mem = mem_ref[...]
    key_ref[...] = jnp.dot(mem, wk_ref[...],
                           preferred_element_type=jnp.float32) + bk_ref[...]
    val_ref[...] = jnp.dot(mem, wv_ref[...],
                           preferred_element_type=jnp.float32) + bv_ref[...]

    csum = jnp.sum(wdg, axis=0, keepdims=True)
    cb = jnp.dot(b1_ref[...], wd_ref[...],
                 preferred_element_type=jnp.float32) + bd_ref[...]
    const_ref[...] = jnp.concatenate(
        [jnp.pad(csum, ((0, 0), (0, 128 - D))),
         jnp.pad(cb, ((0, 0), (0, 128 - D))),
         jnp.zeros((6, 128), jnp.float32)], axis=0)


def _ffn_body(x_ref, wd_aug_ref, wu_aug_ref, key_ref, val_ref, const_ref,
              g2_ref, b2_ref, o_ref):
    H = x_ref.shape[1]
    D = wu_aug_ref.shape[0] - 1

    x = x_ref[...]

    raw = jnp.dot(x, wd_aug_ref[...], preferred_element_type=jnp.float32)
    xw = raw[:, :D]
    m = raw[:, D:D + 1] * (1.0 / H)

    sqsum = jnp.sum(x * x, axis=-1, keepdims=True)
    v = sqsum * (1.0 / H) - m * m
    s = jax.lax.rsqrt(v + _EPS)

    csum = const_ref[0:1, :D]
    cb = const_ref[1:2, :D]
    d = s * (xw - m * csum) + cb

    m2 = jnp.mean(d, axis=-1, keepdims=True)
    dc = d - m2
    v2 = jnp.mean(dc * dc, axis=-1, keepdims=True)
    q = dc * jax.lax.rsqrt(v2 + _EPS) * g2_ref[...] + b2_ref[...]

    logits = jax.lax.dot_general(q, key_ref[...], (((1,), (1,)), ((), ())),
                                 preferred_element_type=jnp.float32)
    logits = logits - jnp.max(logits, axis=-1, keepdims=True)
    e = jnp.exp(logits)
    p = e / jnp.sum(e, axis=-1, keepdims=True)
    mo = jnp.dot(p, val_ref[...], preferred_element_type=jnp.float32)

    m3 = jnp.mean(mo, axis=-1, keepdims=True)
    mc = mo - m3
    v3 = jnp.mean(mc * mc, axis=-1, keepdims=True)
    z = mc * jax.lax.rsqrt(v3 + _EPS)
    z_aug = jnp.concatenate([z, jnp.ones((z.shape[0], 1), jnp.float32)], axis=1)

    o_ref[...] = jnp.dot(z_aug, wu_aug_ref[...], preferred_element_type=jnp.float32)


@functools.partial(jax.jit, static_argnames=("block_rows", "interpret"))
def _run(x2d, g1, b1, W_down, b_down, g2, b2, memory, W_k, b_k, W_v, b_v,
         g3, b3, W_up, b_up, block_rows=1024, interpret=False):
    n, H = x2d.shape
    D = W_down.shape[1]
    M = memory.shape[0]

    def full(a):
        return pl.BlockSpec(a.shape, lambda *_: (0,) * a.ndim)

    prep_ins = (g1.reshape(-1, 1), b1.reshape(1, -1), W_down,
                b_down.reshape(1, -1), memory, W_k, b_k.reshape(1, -1),
                W_v, b_v.reshape(1, -1), g3.reshape(-1, 1), b3.reshape(1, -1),
                W_up, b_up.reshape(1, -1))

    wd_aug, wu_aug, key, val, consts = pl.pallas_call(
        _prep_body,
        out_shape=[jax.ShapeDtypeStruct((H, D + 1), jnp.float32),
                   jax.ShapeDtypeStruct((D + 1, H), jnp.float32),
                   jax.ShapeDtypeStruct((M, D), jnp.float32),
                   jax.ShapeDtypeStruct((M, D), jnp.float32),
                   jax.ShapeDtypeStruct((8, 128), jnp.float32)],
        name="bert_ffn_prep",
        interpret=interpret,
    )(*prep_ins)

    grid = (n // block_rows,)
    main_ins = (wd_aug, wu_aug, key, val, consts,
                g2.reshape(1, -1), b2.reshape(1, -1))

    return pl.pallas_call(
        _ffn_body,
        out_shape=jax.ShapeDtypeStruct((n, H), jnp.float32),
        grid=grid,
        in_specs=[pl.BlockSpec((block_rows, H), lambda i: (i, 0))]
                 + [full(a) for a in main_ins],
        out_specs=pl.BlockSpec((block_rows, H), lambda i: (i, 0)),
        compiler_params=pltpu.CompilerParams(
            dimension_semantics=("arbitrary",),
            vmem_limit_bytes=50 * 1024 * 1024,
        ),
        name="bert_ffn_memory",
        interpret=interpret,
    )(x2d, *main_ins)


def kernel(hidden_states, g1, b1, W_down, b_down, g2, b2, memory, W_k, b_k,
           W_v, b_v, g3, b3, W_up, b_up, layer_id):
    B, S, H = hidden_states.shape
    x2d = hidden_states.reshape(B * S, H)
    out = _run(x2d, g1, b1, W_down, b_down, g2, b2, memory, W_k, b_k,
               W_v, b_v, g3, b3, W_up, b_up)
    return out.reshape(B, S, H)


# speedup vs baseline: 1.1278x; 1.1278x over previous
import functools

import jax
import jax.numpy as jnp
from jax.experimental import pallas as pl
from jax.experimental.pallas import tpu as pltpu

_EPS = 1e-12


def _prep_body(g1c_ref, b1_ref, wd_ref, bd_ref, g2c_ref, b2c_ref,
               mem_ref, wk_ref, bk_ref, wv_ref, bv_ref, g3c_ref, b3_ref,
               wu_ref, bu_ref,
               wd_aug_ref, wu_aug_ref, keyg_t_ref, valc_ref, const_ref):
    H, D = wd_ref.shape

    wdg = wd_ref[...] * g1c_ref[...]
    wdc = wdg - jnp.mean(wdg, axis=1, keepdims=True)
    csum_c = jnp.sum(wdc, axis=0, keepdims=True)
    wdc2 = wdc - csum_c * (1.0 / H)
    wd_aug_ref[...] = jnp.concatenate(
        [wdc2, jnp.ones((H, 1), jnp.float32)], axis=1)

    cb = jnp.dot(b1_ref[...], wd_ref[...],
                 preferred_element_type=jnp.float32) + bd_ref[...]
    cbc = cb - jnp.mean(cb, axis=1, keepdims=True)
    const_ref[...] = jnp.concatenate(
        [jnp.pad(cbc, ((0, 0), (0, 128 - D))),
         jnp.zeros((7, 128), jnp.float32)], axis=0)

    mem = mem_ref[...]
    key = jnp.dot(mem, wk_ref[...],
                  preferred_element_type=jnp.float32) + bk_ref[...]
    keyg_t_ref[...] = key.T * g2c_ref[...]

    val = jnp.dot(mem, wv_ref[...],
                  preferred_element_type=jnp.float32) + bv_ref[...]
    slot_w = jnp.exp(jnp.dot(key, b2c_ref[...],
                             preferred_element_type=jnp.float32))
    valw = val * slot_w
    valc_ref[...] = valw - jnp.mean(valw, axis=1, keepdims=True)

    wug = wu_ref[...] * g3c_ref[...]
    bias_row = jnp.dot(b3_ref[...], wu_ref[...],
                       preferred_element_type=jnp.float32) + bu_ref[...]
    wu_aug_ref[...] = jnp.concatenate([wug, bias_row], axis=0)


def _ffn_body(x_ref, wd_aug_ref, wu_aug_ref, keyg_t_ref, valc_ref,
              const_ref, o_ref):
    H = x_ref.shape[1]
    D = wu_aug_ref.shape[0] - 1

    x = x_ref[...]

    raw = jnp.dot(x, wd_aug_ref[...], preferred_element_type=jnp.float32)
    m = raw[:, D:D + 1] * (1.0 / H)
    sqsum = jnp.sum(x * x, axis=-1, keepdims=True)
    v = sqsum * (1.0 / H) - m * m
    s = jax.lax.rsqrt(v + _EPS)

    cbc = const_ref[0:1, :D]
    dc = s * raw[:, :D] + cbc
    v2 = jnp.mean(dc * dc, axis=-1, keepdims=True)
    qs = dc * jax.lax.rsqrt(v2 + _EPS)

    logits = jnp.dot(qs, keyg_t_ref[...], preferred_element_type=jnp.float32)
    e = jnp.exp(logits)
    mc = jnp.dot(e, valc_ref[...], preferred_element_type=jnp.float32)

    v3 = jnp.mean(mc * mc, axis=-1, keepdims=True)
    z = mc * jax.lax.rsqrt(v3 + _EPS)
    z_aug = jnp.concatenate(
        [z, jnp.ones((z.shape[0], 1), jnp.float32)], axis=1)

    o_ref[...] = jnp.dot(z_aug, wu_aug_ref[...], preferred_element_type=jnp.float32)


@functools.partial(jax.jit, static_argnames=("block_rows", "interpret"))
def _run(x2d, g1, b1, W_down, b_down, g2, b2, memory, W_k, b_k, W_v, b_v,
         g3, b3, W_up, b_up, block_rows=1024, interpret=False):
    n, H = x2d.shape
    D = W_down.shape[1]
    M = memory.shape[0]

    def full(a):
        return pl.BlockSpec(a.shape, lambda *_: (0,) * a.ndim)

    prep_ins = (g1.reshape(-1, 1), b1.reshape(1, -1), W_down,
                b_down.reshape(1, -1), g2.reshape(-1, 1), b2.reshape(-1, 1),
                memory, W_k, b_k.reshape(1, -1), W_v, b_v.reshape(1, -1),
                g3.reshape(-1, 1), b3.reshape(1, -1), W_up, b_up.reshape(1, -1))

    wd_aug, wu_aug, keyg_t, valc, consts = pl.pallas_call(
        _prep_body,
        out_shape=[jax.ShapeDtypeStruct((H, D + 1), jnp.float32),
                   jax.ShapeDtypeStruct((D + 1, H), jnp.float32),
                   jax.ShapeDtypeStruct((D, M), jnp.float32),
                   jax.ShapeDtypeStruct((M, D), jnp.float32),
                   jax.ShapeDtypeStruct((8, 128), jnp.float32)],
        name="bert_ffn_prep",
        interpret=interpret,
    )(*prep_ins)

    grid = (n // block_rows,)
    main_ins = (wd_aug, wu_aug, keyg_t, valc, consts)

    return pl.pallas_call(
        _ffn_body,
        out_shape=jax.ShapeDtypeStruct((n, H), jnp.float32),
        grid=grid,
        in_specs=[pl.BlockSpec((block_rows, H), lambda i: (i, 0))]
                 + [full(a) for a in main_ins],
        out_specs=pl.BlockSpec((block_rows, H), lambda i: (i, 0)),
        compiler_params=pltpu.CompilerParams(
            dimension_semantics=("arbitrary",),
            vmem_limit_bytes=50 * 1024 * 1024,
        ),
        name="bert_ffn_memory",
        interpret=interpret,
    )(x2d, *main_ins)


def kernel(hidden_states, g1, b1, W_down, b_down, g2, b2, memory, W_k, b_k,
           W_v, b_v, g3, b3, W_up, b_up, layer_id):
    B, S, H = hidden_states.shape
    x2d = hidden_states.reshape(B * S, H)
    out = _run(x2d, g1, b1, W_down, b_down, g2, b2, memory, W_k, b_k,
               W_v, b_v, g3, b3, W_up, b_up)
    return out.reshape(B, S, H)
